# baseline (speedup 1.0000x reference)
import functools

import jax
import jax.numpy as jnp
from jax import lax
from jax.experimental import pallas as pl
from jax.experimental.pallas import tpu as pltpu
from jax.experimental.pallas import tpu_sc as plsc

_NE = 1024
_ED = 64
_CC = 0.25
_NI = 16
_PX = 1024
_BLK = 1024
_IPB = 4
_GRID = _NI // _IPB

_NW = 32
_CPW = _ED // _NW


def _tc_body(x_ref, emb_ref, idx_ref, et_ref):
    e = emb_ref[...]
    e2 = e + e
    esq = jnp.sum(e * e, axis=1)

    @pl.when(pl.program_id(0) == 0)
    def _emit_et():
        et_ref[...] = e.T

    for s in range(_IPB):
        xT = x_ref[s]
        mm2 = jax.lax.dot_general(e2, xT, (((1,), (0,)), ((), ())),
                                  preferred_element_type=jnp.float32)
        xsq = jnp.sum(xT * xT, axis=0)
        dist = (xsq[None, :] + esq[:, None]) - mm2
        idx_ref[s, 0, :] = jnp.argmin(dist, axis=0)


def _sc_gather(et_hbm, idx_hbm, x_hbm, q_hbm, part_hbm,
               tbl_v, idx_v, x_v, q_v, acc_v):
    info = plsc.get_sparse_core_info()
    nc = info.num_cores
    wid = lax.axis_index("s") * nc + lax.axis_index("c")
    c0 = wid * _CPW

    pltpu.sync_copy(et_hbm.at[pl.ds(c0 * _NE, _CPW * _NE)], tbl_v)
    pltpu.sync_copy(idx_hbm, idx_v)
    pltpu.sync_copy(x_hbm.at[:, pl.ds(c0, _CPW), :], x_v)

    def chunk(j, acc):
        idxv = idx_v[pl.ds(j * 16, 16)]
        for cl in range(_CPW):
            n = j // (_PX // 16)
            p = j % (_PX // 16)
            qv = plsc.load_gather(tbl_v, [idxv + (cl * _NE)])
            q_v[n, cl, pl.ds(p * 16, 16)] = qv
            xv = x_v[n, cl, pl.ds(p * 16, 16)]
            d = qv - xv
            acc = acc + d * d
        return acc

    acc = lax.fori_loop(0, (_NI * _PX) // 16, chunk,
                        jnp.zeros((16,), jnp.float32))
    acc_v[...] = acc
    pltpu.sync_copy(q_v, q_hbm.at[:, pl.ds(c0, _CPW), :])
    pltpu.sync_copy(acc_v, part_hbm.at[wid])


def kernel(inputs, embedding):
    x3 = inputs.reshape(_NI, _ED, _PX)
    idx3, et = pl.pallas_call(
        _tc_body,
        grid=(_GRID,),
        in_specs=[
            pl.BlockSpec((_IPB, _ED, _BLK), lambda i: (i, 0, 0)),
            pl.BlockSpec((_NE, _ED), lambda i: (0, 0)),
        ],
        out_specs=[
            pl.BlockSpec((_IPB, 1, _BLK), lambda i: (i, 0, 0)),
            pl.BlockSpec((_ED, _NE), lambda i: (0, 0)),
        ],
        out_shape=[
            jax.ShapeDtypeStruct((_NI, 1, _BLK), jnp.int32),
            jax.ShapeDtypeStruct((_ED, _NE), jnp.float32),
        ],
    )(x3, embedding)

    idx_flat = idx3.reshape(_NI * _PX)

    sc = functools.partial(
        pl.kernel,
        mesh=plsc.VectorSubcoreMesh(core_axis_name="c", subcore_axis_name="s"),
        compiler_params=pltpu.CompilerParams(needs_layout_passes=False),
        out_type=[
            jax.ShapeDtypeStruct((_NI, _ED, _PX), jnp.float32),
            jax.ShapeDtypeStruct((_NW, 16), jnp.float32),
        ],
        scratch_types=[
            pltpu.VMEM((_CPW * _NE,), jnp.float32),
            pltpu.VMEM((_NI * _PX,), jnp.int32),
            pltpu.VMEM((_NI, _CPW, _PX), jnp.float32),
            pltpu.VMEM((_NI, _CPW, _PX), jnp.float32),
            pltpu.VMEM((16,), jnp.float32),
        ],
    )(_sc_gather)
    q3, part = sc(et.reshape(_ED * _NE), idx_flat, x3)

    loss = jnp.sum(part) * (_CC / (_NI * _PX * _ED))
    return (loss,
            q3.reshape(_NI, _ED, 32, 32),
            idx_flat[:, None])


# speedup vs baseline: 1.0027x; 1.0027x over previous
import functools

import jax
import jax.numpy as jnp
from jax import lax
from jax.experimental import pallas as pl
from jax.experimental.pallas import tpu as pltpu
from jax.experimental.pallas import tpu_sc as plsc

_NE = 1024
_ED = 64
_CC = 0.25
_NI = 16
_PX = 1024
_BLK = 1024
_IPB = 4
_GRID = _NI // _IPB

_NW = 32
_CPW = _ED // _NW


def _tc_body(x_ref, emb_ref, idx_ref, et_ref):
    e = emb_ref[...]
    e2 = e + e
    esq = jnp.sum(e * e, axis=1)

    @pl.when(pl.program_id(0) == 0)
    def _emit_et():
        et_ref[...] = e.T

    for s in range(_IPB):
        xT = x_ref[s]
        mm2 = jax.lax.dot_general(e2, xT, (((1,), (0,)), ((), ())),
                                  preferred_element_type=jnp.float32)
        xsq = jnp.sum(xT * xT, axis=0)
        dist = (xsq[None, :] + esq[:, None]) - mm2
        idx_ref[s, 0, :] = jnp.argmin(dist, axis=0)


def _sc_gather(et_hbm, idx_hbm, x_hbm, q_hbm, part_hbm,
               tbl_v, idx_v, x_v, q_v, acc_v, sem1, sem2, sem3):
    info = plsc.get_sparse_core_info()
    nc = info.num_cores
    wid = lax.axis_index("s") * nc + lax.axis_index("c")
    c0 = wid * _CPW

    cp1 = pltpu.make_async_copy(et_hbm.at[pl.ds(c0 * _NE, _CPW * _NE)],
                                tbl_v, sem1)
    cp2 = pltpu.make_async_copy(idx_hbm, idx_v, sem2)
    cp3 = pltpu.make_async_copy(x_hbm.at[:, pl.ds(c0, _CPW), :], x_v, sem3)
    cp1.start(); cp2.start(); cp3.start()
    cp1.wait(); cp2.wait(); cp3.wait()

    _UNR = 4

    acc = jnp.zeros((16,), jnp.float32)
    for n in range(_NI):
        def chunk(t, acc, n=n):
            for u in range(_UNR):
                p = t * _UNR + u
                idxv = idx_v[pl.ds(n * _PX + p * 16, 16)]
                for cl in range(_CPW):
                    qv = plsc.load_gather(tbl_v, [idxv + (cl * _NE)])
                    q_v[n, cl, pl.ds(p * 16, 16)] = qv
                    xv = x_v[n, cl, pl.ds(p * 16, 16)]
                    d = qv - xv
                    acc = acc + d * d
            return acc
        acc = lax.fori_loop(0, _PX // (16 * _UNR), chunk, acc)
    acc_v[...] = acc
    pltpu.sync_copy(q_v, q_hbm.at[:, pl.ds(c0, _CPW), :])
    pltpu.sync_copy(acc_v, part_hbm.at[wid])


def kernel(inputs, embedding):
    x3 = inputs.reshape(_NI, _ED, _PX)
    idx3, et = pl.pallas_call(
        _tc_body,
        grid=(_GRID,),
        in_specs=[
            pl.BlockSpec((_IPB, _ED, _BLK), lambda i: (i, 0, 0)),
            pl.BlockSpec((_NE, _ED), lambda i: (0, 0)),
        ],
        out_specs=[
            pl.BlockSpec((_IPB, 1, _BLK), lambda i: (i, 0, 0)),
            pl.BlockSpec((_ED, _NE), lambda i: (0, 0)),
        ],
        out_shape=[
            jax.ShapeDtypeStruct((_NI, 1, _BLK), jnp.int32),
            jax.ShapeDtypeStruct((_ED, _NE), jnp.float32),
        ],
    )(x3, embedding)

    idx_flat = idx3.reshape(_NI * _PX)

    sc = functools.partial(
        pl.kernel,
        mesh=plsc.VectorSubcoreMesh(core_axis_name="c", subcore_axis_name="s"),
        compiler_params=pltpu.CompilerParams(needs_layout_passes=False),
        out_type=[
            jax.ShapeDtypeStruct((_NI, _ED, _PX), jnp.float32),
            jax.ShapeDtypeStruct((_NW, 16), jnp.float32),
        ],
        scratch_types=[
            pltpu.VMEM((_CPW * _NE,), jnp.float32),
            pltpu.VMEM((_NI * _PX,), jnp.int32),
            pltpu.VMEM((_NI, _CPW, _PX), jnp.float32),
            pltpu.VMEM((_NI, _CPW, _PX), jnp.float32),
            pltpu.VMEM((16,), jnp.float32),
            pltpu.SemaphoreType.DMA,
            pltpu.SemaphoreType.DMA,
            pltpu.SemaphoreType.DMA,
        ],
    )(_sc_gather)
    q3, part = sc(et.reshape(_ED * _NE), idx_flat, x3)

    loss = jnp.sum(part) * (_CC / (_NI * _PX * _ED))
    return (loss,
            q3.reshape(_NI, _ED, 32, 32),
            idx_flat[:, None])


# speedup vs baseline: 1.2081x; 1.2048x over previous
import jax
import jax.numpy as jnp
from jax.experimental import pallas as pl
from jax.experimental.pallas import tpu as pltpu

_NE = 1024
_ED = 64
_CC = 0.25
_NI = 16
_PX = 1024
_BLK = 1024
_IPB = 4
_GRID = _NI // _IPB


def _vq_body(x_ref, emb_ref, q_ref, idx_ref, loss_ref):
    e = emb_ref[...]
    e2 = e + e
    esq = jnp.sum(e * e, axis=1)
    part = jnp.float32(0.0)
    for s in range(_IPB):
        xT = x_ref[s].reshape(_ED, _BLK)
        mm2 = jax.lax.dot_general(e2, xT, (((1,), (0,)), ((), ())),
                                  preferred_element_type=jnp.float32)
        xsq = jnp.sum(xT * xT, axis=0)
        dist = (xsq[None, :] + esq[:, None]) - mm2
        idx = jnp.argmin(dist, axis=0)
        idx_ref[s, 0, :] = idx

        ohT = (jax.lax.broadcasted_iota(jnp.int32, (_NE, _BLK), 0)
               == idx[None, :]).astype(jnp.float32)
        qT = jax.lax.dot_general(e, ohT, (((0,), (0,)), ((), ())),
                                 preferred_element_type=jnp.float32)
        q_ref[s] = qT.reshape(_ED, 32, 32)
        part = part + jnp.sum((qT - xT) ** 2)

    @pl.when(pl.program_id(0) == 0)
    def _init():
        loss_ref[0, 0] = 0.0

    loss_ref[0, 0] += part

    @pl.when(pl.program_id(0) == _GRID - 1)
    def _final():
        loss_ref[0, 0] = loss_ref[0, 0] * (_CC / (_NI * _PX * _ED))


def kernel(inputs, embedding):
    q4, idx3, loss = pl.pallas_call(
        _vq_body,
        grid=(_GRID,),
        in_specs=[
            pl.BlockSpec((_IPB, _ED, 32, 32), lambda i: (i, 0, 0, 0)),
            pl.BlockSpec((_NE, _ED), lambda i: (0, 0)),
        ],
        out_specs=[
            pl.BlockSpec((_IPB, _ED, 32, 32), lambda i: (i, 0, 0, 0)),
            pl.BlockSpec((_IPB, 1, _BLK), lambda i: (i, 0, 0)),
            pl.BlockSpec(block_shape=(1, 1), index_map=lambda i: (0, 0),
                         memory_space=pltpu.SMEM),
        ],
        out_shape=[
            jax.ShapeDtypeStruct((_NI, _ED, 32, 32), jnp.float32),
            jax.ShapeDtypeStruct((_NI, 1, _BLK), jnp.int32),
            jax.ShapeDtypeStruct((1, 1), jnp.float32),
        ],
    )(inputs, embedding)
    return (loss[0, 0],
            q4,
            idx3.reshape(_NI * _PX, 1))


# speedup vs baseline: 1.9381x; 1.6043x over previous
import jax
import jax.numpy as jnp
from jax.experimental import pallas as pl
from jax.experimental.pallas import tpu as pltpu

_NE = 1024
_ED = 64
_CC = 0.25
_NI = 16
_PX = 1024
_BLK = 1024
_IPB = 4
_GRID = _NI // _IPB


def _vq_body(x_ref, emb_ref, q_ref, idx_ref, loss_ref):
    e = emb_ref[...]
    e2 = e + e
    esq = jnp.sum(e * e, axis=1)
    part = jnp.float32(0.0)
    for s in range(_IPB):
        xT = x_ref[s]
        mm2 = jax.lax.dot_general(e2, xT, (((1,), (0,)), ((), ())),
                                  preferred_element_type=jnp.float32)
        xsq = jnp.sum(xT * xT, axis=0)
        dist = (xsq[None, :] + esq[:, None]) - mm2
        idx = jnp.argmin(dist, axis=0)
        idx_ref[s, 0, :] = idx

        ohT = (jax.lax.broadcasted_iota(jnp.int32, (_NE, _BLK), 0)
               == idx[None, :]).astype(jnp.float32)
        qT = jax.lax.dot_general(e, ohT, (((0,), (0,)), ((), ())),
                                 preferred_element_type=jnp.float32)
        q_ref[s] = qT
        part = part + jnp.sum((qT - xT) ** 2)

    @pl.when(pl.program_id(0) == 0)
    def _init():
        loss_ref[0, 0] = 0.0

    loss_ref[0, 0] += part

    @pl.when(pl.program_id(0) == _GRID - 1)
    def _final():
        loss_ref[0, 0] = loss_ref[0, 0] * (_CC / (_NI * _PX * _ED))


def kernel(inputs, embedding):
    x3 = inputs.reshape(_NI, _ED, _PX)
    q3, idx3, loss = pl.pallas_call(
        _vq_body,
        grid=(_GRID,),
        in_specs=[
            pl.BlockSpec((_IPB, _ED, _BLK), lambda i: (i, 0, 0)),
            pl.BlockSpec((_NE, _ED), lambda i: (0, 0)),
        ],
        out_specs=[
            pl.BlockSpec((_IPB, _ED, _BLK), lambda i: (i, 0, 0)),
            pl.BlockSpec((_IPB, 1, _BLK), lambda i: (i, 0, 0)),
            pl.BlockSpec(block_shape=(1, 1), index_map=lambda i: (0, 0),
                         memory_space=pltpu.SMEM),
        ],
        out_shape=[
            jax.ShapeDtypeStruct((_NI, _ED, _PX), jnp.float32),
            jax.ShapeDtypeStruct((_NI, 1, _BLK), jnp.int32),
            jax.ShapeDtypeStruct((1, 1), jnp.float32),
        ],
    )(x3, embedding)
    return (loss[0, 0],
            q3.reshape(_NI, _ED, 32, 32),
            idx3.reshape(_NI * _PX, 1))


# speedup vs baseline: 1.9475x; 1.0048x over previous
import jax
import jax.numpy as jnp
from jax.experimental import pallas as pl
from jax.experimental.pallas import tpu as pltpu

_NE = 1024
_ED = 64
_CC = 0.25
_NI = 16
_PX = 1024
_BLK = 1024
_IPB = 8
_GRID = _NI // _IPB


def _vq_body(x_ref, emb_ref, q_ref, idx_ref, loss_ref):
    e = emb_ref[...]
    e2 = e + e
    esq = jnp.sum(e * e, axis=1)
    part = jnp.float32(0.0)
    for s in range(_IPB):
        xT = x_ref[s]
        mm2 = jax.lax.dot_general(e2, xT, (((1,), (0,)), ((), ())),
                                  preferred_element_type=jnp.float32)
        xsq = jnp.sum(xT * xT, axis=0)
        dist = (xsq[None, :] + esq[:, None]) - mm2
        idx = jnp.argmin(dist, axis=0)
        idx_ref[s, 0, :] = idx

        ohT = (jax.lax.broadcasted_iota(jnp.int32, (_NE, _BLK), 0)
               == idx[None, :]).astype(jnp.float32)
        qT = jax.lax.dot_general(e, ohT, (((0,), (0,)), ((), ())),
                                 preferred_element_type=jnp.float32)
        q_ref[s] = qT
        part = part + jnp.sum((qT - xT) ** 2)

    @pl.when(pl.program_id(0) == 0)
    def _init():
        loss_ref[0, 0] = 0.0

    loss_ref[0, 0] += part

    @pl.when(pl.program_id(0) == _GRID - 1)
    def _final():
        loss_ref[0, 0] = loss_ref[0, 0] * (_CC / (_NI * _PX * _ED))


def kernel(inputs, embedding):
    x3 = inputs.reshape(_NI, _ED, _PX)
    q3, idx3, loss = pl.pallas_call(
        _vq_body,
        grid=(_GRID,),
        in_specs=[
            pl.BlockSpec((_IPB, _ED, _BLK), lambda i: (i, 0, 0)),
            pl.BlockSpec((_NE, _ED), lambda i: (0, 0)),
        ],
        out_specs=[
            pl.BlockSpec((_IPB, _ED, _BLK), lambda i: (i, 0, 0)),
            pl.BlockSpec((_IPB, 1, _BLK), lambda i: (i, 0, 0)),
            pl.BlockSpec(block_shape=(1, 1), index_map=lambda i: (0, 0),
                         memory_space=pltpu.SMEM),
        ],
        out_shape=[
            jax.ShapeDtypeStruct((_NI, _ED, _PX), jnp.float32),
            jax.ShapeDtypeStruct((_NI, 1, _BLK), jnp.int32),
            jax.ShapeDtypeStruct((1, 1), jnp.float32),
        ],
    )(x3, embedding)
    return (loss[0, 0],
            q3.reshape(_NI, _ED, 32, 32),
            idx3.reshape(_NI * _PX, 1))
